# baseline (speedup 1.0000x reference)
import functools

import jax
import jax.numpy as jnp
from jax import lax
from jax.experimental import pallas as pl
from jax.experimental.pallas import tpu as pltpu
from jax.experimental.pallas import tpu_sc as plsc

_EMBED = 32
_TILE = 128
_CHUNK_TILES = 8
_GATHER_DN = lax.GatherDimensionNumbers(
    offset_dims=(), collapsed_slice_dims=(0,), start_index_map=(0,))


def _reg_gather(v, idx):
    return lax.gather(v, idx[:, None], _GATHER_DN, slice_sizes=(1,),
                      mode=lax.GatherScatterMode.PROMISE_IN_BOUNDS)


def _body(num_workers, num_cores, xt, w0t, w1t, w2t, of,
          xv0, xv1, qv0, qv1, w0v, w1v, w2v,
          sin0, sin1, sout0, sout1):
    wid = lax.axis_index("s") * num_cores + lax.axis_index("c")

    pltpu.sync_copy(w0t, w0v)
    pltpu.sync_copy(w1t, w1v)
    pltpu.sync_copy(w2t, w2v)

    xvs, qvs = (xv0, xv1), (qv0, qv1)
    sins, souts = (sin0, sin1), (sout0, sout1)
    rows = _CHUNK_TILES * _TILE

    n_tiles = xt.shape[1] // _TILE
    base_cnt = n_tiles // num_workers
    n_extra = n_tiles % num_workers
    cnt = base_cnt + jnp.where(wid < n_extra, 1, 0)
    start = base_cnt * wid + jnp.minimum(wid, n_extra)
    n_full = base_cnt // _CHUNK_TILES
    assert n_full % 2 == 0

    def in_start(c, b):
        col = (start + c * _CHUNK_TILES) * _TILE
        pltpu.async_copy(xt.at[:, pl.ds(col, rows)], xvs[b], sins[b])

    def in_wait(b):
        pltpu.make_async_copy(xt.at[:, pl.ds(0, rows)], xvs[b], sins[b]).wait()

    def out_start(c, b):
        col = (start + c * _CHUNK_TILES) * _TILE
        pltpu.async_copy(qvs[b], of.at[:, pl.ds(col, rows)], souts[b])

    def out_wait(b):
        pltpu.make_async_copy(qvs[b], of.at[:, pl.ds(0, rows)], souts[b]).wait()

    def compute(xv, qv, n_rows):
        for dh in range(2):
            t0 = [w0v[pl.ds((dh * 16 + i) * 16, 16)] for i in range(16)]
            t1 = [w1v[pl.ds((dh * 16 + i) * 16, 16)] for i in range(16)]
            t2 = [w2v[pl.ds((dh * 16 + i) * 16, 16)] for i in range(16)]

            def grp(g, carry):
                for u in range(2):
                    b = (g * 2 + u) * 16
                    x0 = xv[0, pl.ds(b, 16)]
                    x1 = xv[1, pl.ds(b, 16)]
                    x2 = xv[2, pl.ds(b, 16)]
                    for i in range(16):
                        v = (_reg_gather(t0[i], x0) + _reg_gather(t1[i], x1)
                             + _reg_gather(t2[i], x2))
                        qv[dh * 16 + i, pl.ds(b, 16)] = v
                return carry

            lax.fori_loop(0, n_rows // 32, grp, 0)

    in_start(0, 0)
    in_start(1, 1)

    def pair(p, carry):
        for b in range(2):
            c = 2 * p + b

            @pl.when(p >= 1)
            def _():
                out_wait(b)

            in_wait(b)
            compute(xvs[b], qvs[b], rows)
            out_start(c, b)

            @pl.when(c + 2 < n_full)
            def _():
                in_start(c + 2, b)
        return carry

    lax.fori_loop(0, n_full // 2, pair, 0)
    out_wait(0)
    out_wait(1)

    def rem_tile(j, carry):
        tb = start + n_full * _CHUNK_TILES + j
        col = tb * _TILE
        pltpu.sync_copy(xt.at[:, pl.ds(col, _TILE)],
                        xv0.at[:, pl.ds(0, _TILE)])
        compute(xv0, qv0, _TILE)
        pltpu.sync_copy(qv0.at[:, pl.ds(0, _TILE)],
                        of.at[:, pl.ds(col, _TILE)])
        return carry

    lax.fori_loop(0, cnt - n_full * _CHUNK_TILES, rem_tile, 0)


def kernel(x, w0, w1, w2):
    e = x.shape[0]
    info = plsc.get_sparse_core_info()
    nw = info.num_cores * info.num_subcores
    assert e % _TILE == 0

    def tcols(w):
        return jnp.pad(w.T, ((0, 0), (0, 16 - w.shape[0]))).reshape(-1)

    mesh = plsc.VectorSubcoreMesh(core_axis_name="c", subcore_axis_name="s")
    rows_chunk = _CHUNK_TILES * _TILE
    k = pl.kernel(
        functools.partial(_body, nw, info.num_cores),
        out_type=jax.ShapeDtypeStruct((_EMBED, e), jnp.float32),
        mesh=mesh,
        scratch_types=[
            pltpu.VMEM((3, rows_chunk), jnp.int32),
            pltpu.VMEM((3, rows_chunk), jnp.int32),
            pltpu.VMEM((_EMBED, rows_chunk), jnp.float32),
            pltpu.VMEM((_EMBED, rows_chunk), jnp.float32),
            pltpu.VMEM((_EMBED * 16,), jnp.float32),
            pltpu.VMEM((_EMBED * 16,), jnp.float32),
            pltpu.VMEM((_EMBED * 16,), jnp.float32),
            pltpu.SemaphoreType.DMA,
            pltpu.SemaphoreType.DMA,
            pltpu.SemaphoreType.DMA,
            pltpu.SemaphoreType.DMA,
        ],
    )
    out = k(x.T, tcols(w0), tcols(w1), tcols(w2))
    return out.T


# speedup vs baseline: 1.0872x; 1.0872x over previous
import functools

import jax
import jax.numpy as jnp
from jax import lax
from jax.experimental import pallas as pl
from jax.experimental.pallas import tpu as pltpu
from jax.experimental.pallas import tpu_sc as plsc

_EMBED = 32
_TILE = 128
_CHUNK_TILES = 8
_GATHER_DN = lax.GatherDimensionNumbers(
    offset_dims=(), collapsed_slice_dims=(0,), start_index_map=(0,))


def _reg_gather(v, idx):
    return lax.gather(v, idx[:, None], _GATHER_DN, slice_sizes=(1,),
                      mode=lax.GatherScatterMode.PROMISE_IN_BOUNDS)


def _body(num_workers, num_cores, xt, w0t, w1t, w2t, of,
          xv0, xv1, qv0, qv1, w0v, w1v, w2v,
          sin0, sin1, sout0, sout1):
    wid = lax.axis_index("s") * num_cores + lax.axis_index("c")

    pltpu.sync_copy(w0t, w0v)
    pltpu.sync_copy(w1t, w1v)
    pltpu.sync_copy(w2t, w2v)

    xvs, qvs = (xv0, xv1), (qv0, qv1)
    sins, souts = (sin0, sin1), (sout0, sout1)
    rows = _CHUNK_TILES * _TILE

    n_tiles = xt.shape[1] // _TILE
    base_cnt = n_tiles // num_workers
    n_extra = n_tiles % num_workers
    cnt = base_cnt + jnp.where(wid < n_extra, 1, 0)
    start = base_cnt * wid + jnp.minimum(wid, n_extra)
    n_full = base_cnt // _CHUNK_TILES
    assert n_full % 2 == 0

    def in_start(c, b):
        col = (start + c * _CHUNK_TILES) * _TILE
        pltpu.async_copy(xt.at[:, pl.ds(col, rows)], xvs[b], sins[b])

    def in_wait(b):
        pltpu.make_async_copy(xt.at[:, pl.ds(0, rows)], xvs[b], sins[b]).wait()

    def out_start(c, b):
        col = (start + c * _CHUNK_TILES) * _TILE
        pltpu.async_copy(qvs[b], of.at[:, pl.ds(col, rows)], souts[b])

    def out_wait(b):
        pltpu.make_async_copy(qvs[b], of.at[:, pl.ds(0, rows)], souts[b]).wait()

    def compute(xv, qv, n_rows):
        t0 = [w0v[pl.ds(i * 16, 16)] for i in range(16)]
        t1 = [w1v[pl.ds(i * 16, 16)] for i in range(16)]
        t2 = [w2v[pl.ds(i * 16, 16)] for i in range(16)]
        himask = jnp.int32(-65536)

        def grp(g, carry):
            b = g * 16
            x0 = xv[0, pl.ds(b, 16)]
            x1 = xv[1, pl.ds(b, 16)]
            x2 = xv[2, pl.ds(b, 16)]
            for i in range(16):
                p0 = _reg_gather(t0[i], x0)
                p1 = _reg_gather(t1[i], x1)
                p2 = _reg_gather(t2[i], x2)
                vlo = (lax.bitcast_convert_type(p0 << 16, jnp.float32)
                       + lax.bitcast_convert_type(p1 << 16, jnp.float32)
                       + lax.bitcast_convert_type(p2 << 16, jnp.float32))
                vhi = (lax.bitcast_convert_type(p0 & himask, jnp.float32)
                       + lax.bitcast_convert_type(p1 & himask, jnp.float32)
                       + lax.bitcast_convert_type(p2 & himask, jnp.float32))
                qv[i, pl.ds(b, 16)] = vlo
                qv[i + 16, pl.ds(b, 16)] = vhi
            return carry

        lax.fori_loop(0, n_rows // 16, grp, 0)

    in_start(0, 0)
    in_start(1, 1)

    def pair(p, carry):
        for b in range(2):
            c = 2 * p + b

            @pl.when(p >= 1)
            def _():
                out_wait(b)

            in_wait(b)
            compute(xvs[b], qvs[b], rows)
            out_start(c, b)

            @pl.when(c + 2 < n_full)
            def _():
                in_start(c + 2, b)
        return carry

    lax.fori_loop(0, n_full // 2, pair, 0)
    out_wait(0)
    out_wait(1)

    def rem_tile(j, carry):
        tb = start + n_full * _CHUNK_TILES + j
        col = tb * _TILE
        pltpu.sync_copy(xt.at[:, pl.ds(col, _TILE)],
                        xv0.at[:, pl.ds(0, _TILE)])
        compute(xv0, qv0, _TILE)
        pltpu.sync_copy(qv0.at[:, pl.ds(0, _TILE)],
                        of.at[:, pl.ds(col, _TILE)])
        return carry

    lax.fori_loop(0, cnt - n_full * _CHUNK_TILES, rem_tile, 0)


def kernel(x, w0, w1, w2):
    e = x.shape[0]
    info = plsc.get_sparse_core_info()
    nw = info.num_cores * info.num_subcores
    assert e % _TILE == 0

    def tcols(w):
        wt = jnp.pad(w.T, ((0, 0), (0, 16 - w.shape[0])))
        bits = lax.bitcast_convert_type(
            lax.convert_element_type(wt, jnp.bfloat16), jnp.uint16
        ).astype(jnp.int32)
        return (bits[:16] | (bits[16:] << 16)).reshape(-1)

    mesh = plsc.VectorSubcoreMesh(core_axis_name="c", subcore_axis_name="s")
    rows_chunk = _CHUNK_TILES * _TILE
    k = pl.kernel(
        functools.partial(_body, nw, info.num_cores),
        out_type=jax.ShapeDtypeStruct((_EMBED, e), jnp.float32),
        mesh=mesh,
        scratch_types=[
            pltpu.VMEM((3, rows_chunk), jnp.int32),
            pltpu.VMEM((3, rows_chunk), jnp.int32),
            pltpu.VMEM((_EMBED, rows_chunk), jnp.float32),
            pltpu.VMEM((_EMBED, rows_chunk), jnp.float32),
            pltpu.VMEM((16 * 16,), jnp.int32),
            pltpu.VMEM((16 * 16,), jnp.int32),
            pltpu.VMEM((16 * 16,), jnp.int32),
            pltpu.SemaphoreType.DMA,
            pltpu.SemaphoreType.DMA,
            pltpu.SemaphoreType.DMA,
            pltpu.SemaphoreType.DMA,
        ],
    )
    out = k(x.T, tcols(w0), tcols(w1), tcols(w2))
    return out.T


# speedup vs baseline: 1.1367x; 1.0455x over previous
import functools

import jax
import jax.numpy as jnp
from jax import lax
from jax.experimental import pallas as pl
from jax.experimental.pallas import tpu as pltpu
from jax.experimental.pallas import tpu_sc as plsc

_EMBED = 32
_TILE = 128
_CHUNK_TILES = 12
_GATHER_DN = lax.GatherDimensionNumbers(
    offset_dims=(), collapsed_slice_dims=(0,), start_index_map=(0,))


def _reg_gather(v, idx):
    return lax.gather(v, idx[:, None], _GATHER_DN, slice_sizes=(1,),
                      mode=lax.GatherScatterMode.PROMISE_IN_BOUNDS)


def _body(num_workers, num_cores, xt, w0t, w1t, w2t, of,
          xv0, xv1, qv0, qv1, w0v, w1v, w2v,
          sin0, sin1, sout0, sout1):
    wid = lax.axis_index("s") * num_cores + lax.axis_index("c")

    pltpu.sync_copy(w0t, w0v)
    pltpu.sync_copy(w1t, w1v)
    pltpu.sync_copy(w2t, w2v)

    xvs, qvs = (xv0, xv1), (qv0, qv1)
    sins, souts = (sin0, sin1), (sout0, sout1)
    rows = _CHUNK_TILES * _TILE

    n_tiles = xt.shape[1] // _TILE
    base_cnt = n_tiles // num_workers
    n_extra = n_tiles % num_workers
    cnt = base_cnt + jnp.where(wid < n_extra, 1, 0)
    start = base_cnt * wid + jnp.minimum(wid, n_extra)
    n_full = base_cnt // _CHUNK_TILES
    assert n_full % 2 == 0

    def in_start(c, b):
        col = (start + c * _CHUNK_TILES) * _TILE
        pltpu.async_copy(xt.at[:, pl.ds(col, rows)], xvs[b], sins[b])

    def in_wait(b):
        pltpu.make_async_copy(xt.at[:, pl.ds(0, rows)], xvs[b], sins[b]).wait()

    def out_start(c, b):
        col = (start + c * _CHUNK_TILES) * _TILE
        pltpu.async_copy(qvs[b], of.at[:, pl.ds(col, rows)], souts[b])

    def out_wait(b):
        pltpu.make_async_copy(qvs[b], of.at[:, pl.ds(0, rows)], souts[b]).wait()

    def compute(xv, qv, n_rows):
        for dh in range(2):
            t0 = [w0v[pl.ds((dh * 16 + i) * 16, 16)] for i in range(16)]
            t1 = [w1v[pl.ds((dh * 16 + i) * 16, 16)] for i in range(16)]
            t2 = [w2v[pl.ds((dh * 16 + i) * 16, 16)] for i in range(16)]

            def grp(g, carry):
                b = g * 16
                x0 = xv[0, pl.ds(b, 16)]
                x1 = xv[1, pl.ds(b, 16)]
                x2 = xv[2, pl.ds(b, 16)]
                for i in range(16):
                    v = (_reg_gather(t0[i], x0) + _reg_gather(t1[i], x1)
                         + _reg_gather(t2[i], x2))
                    qv[dh * 16 + i, pl.ds(b, 16)] = v
                return carry

            lax.fori_loop(0, n_rows // 16, grp, 0)

    in_start(0, 0)
    in_start(1, 1)

    def pair(p, carry):
        for b in range(2):
            c = 2 * p + b

            @pl.when(p >= 1)
            def _():
                out_wait(b)

            in_wait(b)
            compute(xvs[b], qvs[b], rows)
            out_start(c, b)

            @pl.when(c + 2 < n_full)
            def _():
                in_start(c + 2, b)
        return carry

    lax.fori_loop(0, n_full // 2, pair, 0)
    out_wait(0)
    out_wait(1)

    def rem_tile(j, carry):
        tb = start + n_full * _CHUNK_TILES + j
        col = tb * _TILE
        pltpu.sync_copy(xt.at[:, pl.ds(col, _TILE)],
                        xv0.at[:, pl.ds(0, _TILE)])
        compute(xv0, qv0, _TILE)
        pltpu.sync_copy(qv0.at[:, pl.ds(0, _TILE)],
                        of.at[:, pl.ds(col, _TILE)])
        return carry

    lax.fori_loop(0, cnt - n_full * _CHUNK_TILES, rem_tile, 0)


def kernel(x, w0, w1, w2):
    e = x.shape[0]
    info = plsc.get_sparse_core_info()
    nw = info.num_cores * info.num_subcores
    assert e % _TILE == 0

    def tcols(w):
        return jnp.pad(w.T, ((0, 0), (0, 16 - w.shape[0]))).reshape(-1)

    mesh = plsc.VectorSubcoreMesh(core_axis_name="c", subcore_axis_name="s")
    rows_chunk = _CHUNK_TILES * _TILE
    k = pl.kernel(
        functools.partial(_body, nw, info.num_cores),
        out_type=jax.ShapeDtypeStruct((_EMBED, e), jnp.float32),
        mesh=mesh,
        scratch_types=[
            pltpu.VMEM((3, rows_chunk), jnp.int32),
            pltpu.VMEM((3, rows_chunk), jnp.int32),
            pltpu.VMEM((_EMBED, rows_chunk), jnp.float32),
            pltpu.VMEM((_EMBED, rows_chunk), jnp.float32),
            pltpu.VMEM((_EMBED * 16,), jnp.float32),
            pltpu.VMEM((_EMBED * 16,), jnp.float32),
            pltpu.VMEM((_EMBED * 16,), jnp.float32),
            pltpu.SemaphoreType.DMA,
            pltpu.SemaphoreType.DMA,
            pltpu.SemaphoreType.DMA,
            pltpu.SemaphoreType.DMA,
        ],
    )
    out = k(x.T, tcols(w0), tcols(w1), tcols(w2))
    return out.T
